# baseline (speedup 1.0000x reference)
import math

import jax
import jax.numpy as jnp
import numpy as np
from jax.experimental import pallas as pl

N = 32768
D = 3
T = 16
RESOLUTION = 64
RADIUS = 1.0
SCALE = 8.0
MAX_CHANNELS = 4
BATCH_LEN = 16
NUM_SEG = BATCH_LEN * MAX_CHANNELS

CHUNK = 4096
NUM_BLOCKS = N // CHUNK

_A = SCALE * math.log2(math.e)
_CLAMP = 126.0

_LIN = np.linspace(-RADIUS, RADIUS, RESOLUTION).astype(np.float64)
_S = (np.arange(T)[:, None] == (np.arange(T * RESOLUTION)[None, :] % T)).astype(
    np.float64
) * np.exp2(-_A * _LIN)[np.arange(T * RESOLUTION) // T][None, :]
_S = _S.astype(np.float32)


def _ect_kernel(x_ref, v_ref, s_ref, index_ref, chan_ref, out_ref):
    step = pl.program_id(0)

    x = x_ref[...]
    v2 = _A * v_ref[...]
    m = jnp.dot(x, v2, preferred_element_type=jnp.float32)
    m = jnp.clip(m, -_CLAMP, _CLAMP)
    e = jnp.exp2(m).astype(jnp.bfloat16)

    p32 = jnp.dot(e, s_ref[...], preferred_element_type=jnp.float32)
    p = p32.astype(jnp.bfloat16)
    one = jnp.bfloat16(1.0)
    sigb = one / (one + p)

    idx = MAX_CHANNELS * index_ref[0] + chan_ref[0]
    seg = jax.lax.broadcasted_iota(jnp.int32, (NUM_SEG, CHUNK), 0)
    onehot = (idx == seg).astype(jnp.bfloat16)

    contrib = jnp.dot(onehot, sigb, preferred_element_type=jnp.float32)

    @pl.when(step == 0)
    def _init():
        out_ref[...] = contrib

    @pl.when(step > 0)
    def _acc():
        out_ref[...] = out_ref[...] + contrib

    @pl.when(step == NUM_BLOCKS - 1)
    def _normalize():
        acc = out_ref[...]
        mx = jnp.max(acc, axis=1, keepdims=True)
        mx = jnp.where(mx == 0.0, 1.0, mx)
        out_ref[...] = acc / mx


@jax.jit
def kernel(x, v, index, channels):
    index3 = index.reshape(NUM_BLOCKS, 1, CHUNK)
    chan3 = channels.reshape(NUM_BLOCKS, 1, CHUNK)
    s = jnp.asarray(_S, dtype=jnp.bfloat16)

    out = pl.pallas_call(
        _ect_kernel,
        grid=(NUM_BLOCKS,),
        in_specs=[
            pl.BlockSpec((CHUNK, D), lambda i: (i, 0)),
            pl.BlockSpec((D, T), lambda i: (0, 0)),
            pl.BlockSpec((T, T * RESOLUTION), lambda i: (0, 0)),
            pl.BlockSpec((1, 1, CHUNK), lambda i: (i, 0, 0)),
            pl.BlockSpec((1, 1, CHUNK), lambda i: (i, 0, 0)),
        ],
        out_specs=pl.BlockSpec((NUM_SEG, T * RESOLUTION), lambda i: (0, 0)),
        out_shape=jax.ShapeDtypeStruct((NUM_SEG, T * RESOLUTION), jnp.float32),
    )(x, v, s, index3, chan3)

    return out.reshape(BATCH_LEN, MAX_CHANNELS, RESOLUTION, T)


# speedup vs baseline: 1.1678x; 1.1678x over previous
import math

import jax
import jax.numpy as jnp
import numpy as np
from jax.experimental import pallas as pl

N = 32768
D = 3
T = 16
RESOLUTION = 64
RADIUS = 1.0
SCALE = 8.0
MAX_CHANNELS = 4
BATCH_LEN = 16
NUM_SEG = BATCH_LEN * MAX_CHANNELS

CHUNK = 4096
NUM_BLOCKS = N // CHUNK

_A = SCALE * math.log2(math.e)
_CLAMP = 126.0

_LIN = np.linspace(-RADIUS, RADIUS, RESOLUTION).astype(np.float64)
_S = (np.arange(T)[:, None] == (np.arange(T * RESOLUTION)[None, :] % T)).astype(
    np.float64
) * np.exp2(-_A * _LIN)[np.arange(T * RESOLUTION) // T][None, :]
_S = _S.astype(np.float32)


def _ect_kernel(x_ref, v_ref, s_ref, index_ref, chan_ref, out_ref):
    step = pl.program_id(0)

    x = x_ref[...]
    v2 = _A * v_ref[...]
    m = jnp.dot(x, v2, preferred_element_type=jnp.float32)
    m = jnp.clip(m, -_CLAMP, _CLAMP)
    e = jnp.exp2(m).astype(jnp.bfloat16)

    p = jnp.dot(e, s_ref[...], preferred_element_type=jnp.float32)
    sigb = (1.0 / (1.0 + p)).astype(jnp.bfloat16)

    idx = MAX_CHANNELS * index_ref[0] + chan_ref[0]
    seg = jax.lax.broadcasted_iota(jnp.int32, (NUM_SEG, CHUNK), 0)
    onehot = (idx == seg).astype(jnp.bfloat16)

    contrib = jnp.dot(onehot, sigb, preferred_element_type=jnp.float32)

    @pl.when(step == 0)
    def _init():
        out_ref[...] = contrib

    @pl.when(step > 0)
    def _acc():
        out_ref[...] = out_ref[...] + contrib

    @pl.when(step == NUM_BLOCKS - 1)
    def _normalize():
        acc = out_ref[...]
        mx = jnp.max(acc, axis=1, keepdims=True)
        mx = jnp.where(mx == 0.0, 1.0, mx)
        out_ref[...] = acc / mx


@jax.jit
def kernel(x, v, index, channels):
    index3 = index.reshape(NUM_BLOCKS, 1, CHUNK)
    chan3 = channels.reshape(NUM_BLOCKS, 1, CHUNK)
    s = jnp.asarray(_S, dtype=jnp.bfloat16)

    out = pl.pallas_call(
        _ect_kernel,
        grid=(NUM_BLOCKS,),
        in_specs=[
            pl.BlockSpec((CHUNK, D), lambda i: (i, 0)),
            pl.BlockSpec((D, T), lambda i: (0, 0)),
            pl.BlockSpec((T, T * RESOLUTION), lambda i: (0, 0)),
            pl.BlockSpec((1, 1, CHUNK), lambda i: (i, 0, 0)),
            pl.BlockSpec((1, 1, CHUNK), lambda i: (i, 0, 0)),
        ],
        out_specs=pl.BlockSpec((NUM_SEG, T * RESOLUTION), lambda i: (0, 0)),
        out_shape=jax.ShapeDtypeStruct((NUM_SEG, T * RESOLUTION), jnp.float32),
    )(x, v, s, index3, chan3)

    return out.reshape(BATCH_LEN, MAX_CHANNELS, RESOLUTION, T)


# speedup vs baseline: 1.1695x; 1.0015x over previous
import math

import jax
import jax.numpy as jnp
import numpy as np
from jax.experimental import pallas as pl

N = 32768
D = 3
T = 16
RESOLUTION = 64
RADIUS = 1.0
SCALE = 8.0
MAX_CHANNELS = 4
BATCH_LEN = 16
NUM_SEG = BATCH_LEN * MAX_CHANNELS

CHUNK = 8192
NUM_BLOCKS = N // CHUNK

_A = SCALE * math.log2(math.e)
_CLAMP = 126.0

_LIN = np.linspace(-RADIUS, RADIUS, RESOLUTION).astype(np.float64)
_S = (np.arange(T)[:, None] == (np.arange(T * RESOLUTION)[None, :] % T)).astype(
    np.float64
) * np.exp2(-_A * _LIN)[np.arange(T * RESOLUTION) // T][None, :]
_S = _S.astype(np.float32)


def _ect_kernel(x_ref, v_ref, s_ref, index_ref, chan_ref, out_ref):
    step = pl.program_id(0)

    x = x_ref[...]
    v2 = _A * v_ref[...]
    m = jnp.dot(x, v2, preferred_element_type=jnp.float32)
    m = jnp.clip(m, -_CLAMP, _CLAMP)
    e = jnp.exp2(m).astype(jnp.bfloat16)

    p = jnp.dot(e, s_ref[...], preferred_element_type=jnp.float32)
    sigb = (1.0 / (1.0 + p)).astype(jnp.bfloat16)

    idx = MAX_CHANNELS * index_ref[0] + chan_ref[0]
    seg = jax.lax.broadcasted_iota(jnp.int32, (NUM_SEG, CHUNK), 0)
    onehot = (idx == seg).astype(jnp.bfloat16)

    contrib = jnp.dot(onehot, sigb, preferred_element_type=jnp.float32)

    @pl.when(step == 0)
    def _init():
        out_ref[...] = contrib

    @pl.when(step > 0)
    def _acc():
        out_ref[...] = out_ref[...] + contrib

    @pl.when(step == NUM_BLOCKS - 1)
    def _normalize():
        acc = out_ref[...]
        mx = jnp.max(acc, axis=1, keepdims=True)
        mx = jnp.where(mx == 0.0, 1.0, mx)
        out_ref[...] = acc / mx


@jax.jit
def kernel(x, v, index, channels):
    index3 = index.reshape(NUM_BLOCKS, 1, CHUNK)
    chan3 = channels.reshape(NUM_BLOCKS, 1, CHUNK)
    s = jnp.asarray(_S, dtype=jnp.bfloat16)

    out = pl.pallas_call(
        _ect_kernel,
        grid=(NUM_BLOCKS,),
        in_specs=[
            pl.BlockSpec((CHUNK, D), lambda i: (i, 0)),
            pl.BlockSpec((D, T), lambda i: (0, 0)),
            pl.BlockSpec((T, T * RESOLUTION), lambda i: (0, 0)),
            pl.BlockSpec((1, 1, CHUNK), lambda i: (i, 0, 0)),
            pl.BlockSpec((1, 1, CHUNK), lambda i: (i, 0, 0)),
        ],
        out_specs=pl.BlockSpec((NUM_SEG, T * RESOLUTION), lambda i: (0, 0)),
        out_shape=jax.ShapeDtypeStruct((NUM_SEG, T * RESOLUTION), jnp.float32),
    )(x, v, s, index3, chan3)

    return out.reshape(BATCH_LEN, MAX_CHANNELS, RESOLUTION, T)
